# baseline (speedup 1.0000x reference)
import functools

import jax
import jax.numpy as jnp
from jax import lax
from jax.experimental import pallas as pl
from jax.experimental.pallas import tpu as pltpu
from jax.experimental.pallas import tpu_sc as plsc

_N = 10000
_E = 320000
_D = 128
_NC = 2
_NS = 16
_NW = _NC * _NS
_NPAD = 10112
_ROWS_PER_TILE = _NPAD // _NS
_EPW = _E // _NW
_CH = 80
_NCH = _EPW // _CH
_G = _CH // 16
_RROWS = _NPAD // _NW
_WORDS = _RROWS * _D


def _matmul(x, w):
    m = x.shape[0]
    bm = 2000

    def body(x_ref, w_ref, o_ref):
        o_ref[...] = jnp.dot(x_ref[...], w_ref[...],
                             preferred_element_type=jnp.float32)

    return pl.pallas_call(
        body,
        grid=(m // bm,),
        in_specs=[
            pl.BlockSpec((bm, _D), lambda i: (i, 0)),
            pl.BlockSpec((_D, _D), lambda i: (0, 0)),
        ],
        out_specs=pl.BlockSpec((bm, _D), lambda i: (i, 0)),
        out_shape=jax.ShapeDtypeStruct((m, _D), jnp.float32),
    )(x, w)


def _hop(table, src, dst, w, zeros):
    mesh = plsc.VectorSubcoreMesh(
        core_axis_name="c", subcore_axis_name="s",
        num_cores=_NC, num_subcores=_NS)

    nslot = 4
    nbank = 8

    @functools.partial(
        pl.kernel,
        mesh=mesh,
        compiler_params=pltpu.CompilerParams(needs_layout_passes=False),
        out_type=jax.ShapeDtypeStruct((_NC, _NPAD, _D), jnp.float32),
        scratch_types=(
            [pltpu.VMEM((_CH, _D), jnp.float32)] * nslot
            + [pltpu.VMEM((_CH,), jnp.int32)] * nbank
            + [pltpu.VMEM((_CH,), jnp.int32)] * nbank
            + [pltpu.VMEM((_CH,), jnp.float32)] * nbank
            + [pltpu.VMEM_SHARED((_NPAD, _D), jnp.float32)]
            + [pltpu.SemaphoreType.DMA] * nslot
            + [pltpu.SemaphoreType.DMA] * nslot
            + [pltpu.SemaphoreType.DMA] * nbank
        ),
    )
    def k(table_hbm, src_hbm, dst_hbm, w_hbm, zeros_hbm, out_hbm, *rest):
        slots = rest[0:nslot]
        sbs = rest[nslot:nslot + nbank]
        dbs = rest[nslot + nbank:nslot + 2 * nbank]
        wbs = rest[nslot + 2 * nbank:nslot + 3 * nbank]
        acc = rest[nslot + 3 * nbank]
        gsem = rest[nslot + 3 * nbank + 1:2 * nslot + 3 * nbank + 1]
        ssem = rest[2 * nslot + 3 * nbank + 1:3 * nslot + 3 * nbank + 1]
        isem = rest[3 * nslot + 3 * nbank + 1:]

        cid = lax.axis_index("c")
        sid = lax.axis_index("s")
        wid = sid * _NC + cid
        r0 = sid * _ROWS_PER_TILE
        cbase = wid * _NCH
        last = _NCH - 1

        pltpu.sync_copy(zeros_hbm.at[pl.ds(r0, _ROWS_PER_TILE)],
                        acc.at[pl.ds(r0, _ROWS_PER_TILE)])
        plsc.subcore_barrier()

        def idx_start(c, b):
            pltpu.async_copy(src_hbm.at[cbase + c], sbs[b], isem[b])
            pltpu.async_copy(dst_hbm.at[cbase + c], dbs[b], isem[b])
            pltpu.async_copy(w_hbm.at[cbase + c], wbs[b], isem[b])

        def idx_wait(c, b):
            pltpu.make_async_copy(src_hbm.at[cbase + c], sbs[b],
                                  isem[b]).wait()
            pltpu.make_async_copy(dst_hbm.at[cbase + c], dbs[b],
                                  isem[b]).wait()
            pltpu.make_async_copy(w_hbm.at[cbase + c], wbs[b],
                                  isem[b]).wait()

        def gather(b, s):
            pltpu.async_copy(table_hbm.at[sbs[b]], slots[s], gsem[s])

        def gwait(b, s):
            pltpu.make_async_copy(table_hbm.at[sbs[b]], slots[s],
                                  gsem[s]).wait()

        def scat(b, s):
            pltpu.async_copy(slots[s], acc.at[dbs[b]], ssem[s], add=True)

        def swait(b, s):
            pltpu.make_async_copy(slots[s], acc.at[dbs[b]], ssem[s]).wait()

        def scale(s, b):
            slot, wb = slots[s], wbs[b]

            def edge(e, carry):
                wsp = plsc.load_gather(wb, [jnp.full((16,), e, jnp.int32)])
                vals = [slot[e, pl.ds(j * 16, 16)] for j in range(_D // 16)]
                for j in range(_D // 16):
                    slot[e, pl.ds(j * 16, 16)] = vals[j] * wsp
                return carry

            lax.fori_loop(0, _CH, edge, 0, unroll=4)

        def side(c, k4):
            s = k4 % nslot
            b = k4 % nbank
            gwait(b, s)
            scale(s, b)
            scat(b, s)

            @pl.when(c >= 1)
            def _():
                swait((b - 1) % nbank, (s - 1) % nslot)

            @pl.when(c + 3 <= last)
            def _():
                idx_wait(c + 3, (b + 3) % nbank)
                gather((b + 3) % nbank, (s + 3) % nslot)

            @pl.when(c + 7 <= last)
            def _():
                idx_start(c + 7, (b + 7) % nbank)

        for c in range(7):
            idx_start(c, c)
        for c in range(3):
            idx_wait(c, c)
            gather(c, c)

        def oct_(j, carry):
            c8 = 8 * j
            for k4 in range(8):
                side(c8 + k4, k4)
            return carry

        lax.fori_loop(0, 15, oct_, 0)
        for c in range(120, 125):
            side(c, c % 8)

        swait(124 % nbank, 124 % nslot)

        plsc.subcore_barrier()
        pltpu.sync_copy(acc.at[pl.ds(r0, _ROWS_PER_TILE)],
                        out_hbm.at[cid, pl.ds(r0, _ROWS_PER_TILE)])

    return k(table, src, dst, w, zeros)


def _reduce(parts_flat, bias):
    mesh = plsc.VectorSubcoreMesh(
        core_axis_name="c", subcore_axis_name="s",
        num_cores=_NC, num_subcores=_NS)

    @functools.partial(
        pl.kernel,
        mesh=mesh,
        out_type=jax.ShapeDtypeStruct((_NPAD * _D,), jnp.float32),
        scratch_types=[
            pltpu.VMEM((_WORDS,), jnp.float32),
            pltpu.VMEM((_WORDS,), jnp.float32),
            pltpu.VMEM((_D,), jnp.float32),
        ],
    )
    def k(p_hbm, b_hbm, out_hbm, a_v, b_v, bias_v):
        cid = lax.axis_index("c")
        sid = lax.axis_index("s")
        wid = sid * _NC + cid
        base = wid * _WORDS
        pltpu.sync_copy(p_hbm.at[0, pl.ds(base, _WORDS)], a_v)
        pltpu.sync_copy(p_hbm.at[1, pl.ds(base, _WORDS)], b_v)
        pltpu.sync_copy(b_hbm, bias_v)
        bias_vecs = [bias_v[pl.ds(j * 16, 16)] for j in range(_D // 16)]

        def row(r, carry):
            rb = r * _D
            for j in range(_D // 16):
                sl = pl.ds(rb + j * 16, 16)
                a_v[sl] = a_v[sl] + b_v[sl] + bias_vecs[j]
            return carry

        lax.fori_loop(0, _RROWS, row, 0)
        pltpu.sync_copy(a_v, out_hbm.at[pl.ds(base, _WORDS)])

    return k(parts_flat, bias)


def kernel(input, edge_index, edge_weight, W, b):
    dst = edge_index[0].reshape(_E // _CH, _CH)
    src = edge_index[1].reshape(_E // _CH, _CH)
    ew = edge_weight.reshape(_E // _CH, _CH)
    zeros = jnp.zeros((_NPAD, _D), jnp.float32)
    xw = _matmul(input, W)
    p1 = _hop(xw, src, dst, ew, zeros)
    agg = _reduce(p1.reshape(_NC, _NPAD * _D),
                  jnp.zeros((_D,), jnp.float32)).reshape(_NPAD, _D)
    p2 = _hop(agg, src, dst, ew, zeros)
    out = _reduce(p2.reshape(_NC, _NPAD * _D), b).reshape(_NPAD, _D)
    return out[:_N]


# speedup vs baseline: 1.0556x; 1.0556x over previous
import functools

import jax
import jax.numpy as jnp
from jax import lax
from jax.experimental import pallas as pl
from jax.experimental.pallas import tpu as pltpu
from jax.experimental.pallas import tpu_sc as plsc

_N = 10000
_E = 320000
_D = 128
_NC = 2
_NS = 16
_NW = _NC * _NS
_NPAD = 10112
_ROWS_PER_TILE = _NPAD // _NS
_EPW = _E // _NW
_CH = 80
_NCH = _EPW // _CH
_G = _CH // 16
_RROWS = _NPAD // _NW
_WORDS = _RROWS * _D


def _matmul(x, w):
    m = x.shape[0]
    bm = 2000

    def body(x_ref, w_ref, o_ref):
        o_ref[...] = jnp.dot(x_ref[...], w_ref[...],
                             preferred_element_type=jnp.float32)

    return pl.pallas_call(
        body,
        grid=(m // bm,),
        in_specs=[
            pl.BlockSpec((bm, _D), lambda i: (i, 0)),
            pl.BlockSpec((_D, _D), lambda i: (0, 0)),
        ],
        out_specs=pl.BlockSpec((bm, _D), lambda i: (i, 0)),
        out_shape=jax.ShapeDtypeStruct((m, _D), jnp.float32),
    )(x, w)


def _hop(table, edges, w, zeros):
    mesh = plsc.VectorSubcoreMesh(
        core_axis_name="c", subcore_axis_name="s",
        num_cores=_NC, num_subcores=_NS)

    nslot = 4
    nbank = 8

    @functools.partial(
        pl.kernel,
        mesh=mesh,
        compiler_params=pltpu.CompilerParams(needs_layout_passes=False),
        out_type=jax.ShapeDtypeStruct((_NC, _NPAD, _D), jnp.float32),
        scratch_types=(
            [pltpu.VMEM((_CH, _D), jnp.float32)] * nslot
            + [pltpu.VMEM((_CH,), jnp.int32)] * nbank
            + [pltpu.VMEM((_CH,), jnp.int32)] * nbank
            + [pltpu.VMEM((_CH,), jnp.float32)] * nbank
            + [pltpu.VMEM_SHARED((_NPAD, _D), jnp.float32)]
            + [pltpu.SemaphoreType.DMA] * nslot
            + [pltpu.SemaphoreType.DMA] * nslot
            + [pltpu.SemaphoreType.DMA] * nbank
        ),
    )
    def k(table_hbm, edges_hbm, w_hbm, zeros_hbm, out_hbm, *rest):
        slots = rest[0:nslot]
        sbs = rest[nslot:nslot + nbank]
        dbs = rest[nslot + nbank:nslot + 2 * nbank]
        wbs = rest[nslot + 2 * nbank:nslot + 3 * nbank]
        acc = rest[nslot + 3 * nbank]
        gsem = rest[nslot + 3 * nbank + 1:2 * nslot + 3 * nbank + 1]
        ssem = rest[2 * nslot + 3 * nbank + 1:3 * nslot + 3 * nbank + 1]
        isem = rest[3 * nslot + 3 * nbank + 1:]

        cid = lax.axis_index("c")
        sid = lax.axis_index("s")
        wid = sid * _NC + cid
        r0 = sid * _ROWS_PER_TILE
        cbase = wid * _NCH
        last = _NCH - 1

        pltpu.sync_copy(zeros_hbm.at[pl.ds(r0, _ROWS_PER_TILE)],
                        acc.at[pl.ds(r0, _ROWS_PER_TILE)])
        plsc.subcore_barrier()

        def idx_start(c, b):
            pltpu.async_copy(edges_hbm.at[1, cbase + c], sbs[b], isem[b])
            pltpu.async_copy(edges_hbm.at[0, cbase + c], dbs[b], isem[b])
            pltpu.async_copy(w_hbm.at[cbase + c], wbs[b], isem[b])

        def idx_wait(c, b):
            pltpu.make_async_copy(edges_hbm.at[1, cbase + c], sbs[b],
                                  isem[b]).wait()
            pltpu.make_async_copy(edges_hbm.at[0, cbase + c], dbs[b],
                                  isem[b]).wait()
            pltpu.make_async_copy(w_hbm.at[cbase + c], wbs[b],
                                  isem[b]).wait()

        def gather(b, s):
            pltpu.async_copy(table_hbm.at[sbs[b]], slots[s], gsem[s])

        def gwait(b, s):
            pltpu.make_async_copy(table_hbm.at[sbs[b]], slots[s],
                                  gsem[s]).wait()

        def scat(b, s):
            pltpu.async_copy(slots[s], acc.at[dbs[b]], ssem[s], add=True)

        def swait(b, s):
            pltpu.make_async_copy(slots[s], acc.at[dbs[b]], ssem[s]).wait()

        def scale(s, b):
            slot, wb = slots[s], wbs[b]

            def edge(e, carry):
                wsp = plsc.load_gather(wb, [jnp.full((16,), e, jnp.int32)])
                vals = [slot[e, pl.ds(j * 16, 16)] for j in range(_D // 16)]
                for j in range(_D // 16):
                    slot[e, pl.ds(j * 16, 16)] = vals[j] * wsp
                return carry

            lax.fori_loop(0, _CH, edge, 0, unroll=2)

        def side(c, k4):
            s = k4 % nslot
            b = k4 % nbank
            gwait(b, s)
            scale(s, b)
            scat(b, s)

            @pl.when(c >= 1)
            def _():
                swait((b - 1) % nbank, (s - 1) % nslot)

            @pl.when(c + 3 <= last)
            def _():
                idx_wait(c + 3, (b + 3) % nbank)
                gather((b + 3) % nbank, (s + 3) % nslot)

            @pl.when(c + 7 <= last)
            def _():
                idx_start(c + 7, (b + 7) % nbank)

        for c in range(7):
            idx_start(c, c)
        for c in range(3):
            idx_wait(c, c)
            gather(c, c)

        def oct_(j, carry):
            c8 = 8 * j
            for k4 in range(8):
                side(c8 + k4, k4)
            return carry

        lax.fori_loop(0, 15, oct_, 0)
        for c in range(120, 125):
            side(c, c % 8)

        swait(124 % nbank, 124 % nslot)

        plsc.subcore_barrier()
        pltpu.sync_copy(acc.at[pl.ds(r0, _ROWS_PER_TILE)],
                        out_hbm.at[cid, pl.ds(r0, _ROWS_PER_TILE)])

    return k(table, edges, w, zeros)


def _reduce(parts_flat, bias):
    mesh = plsc.VectorSubcoreMesh(
        core_axis_name="c", subcore_axis_name="s",
        num_cores=_NC, num_subcores=_NS)

    @functools.partial(
        pl.kernel,
        mesh=mesh,
        out_type=jax.ShapeDtypeStruct((_NPAD * _D,), jnp.float32),
        scratch_types=[
            pltpu.VMEM((_WORDS,), jnp.float32),
            pltpu.VMEM((_WORDS,), jnp.float32),
            pltpu.VMEM((_D,), jnp.float32),
        ],
    )
    def k(p_hbm, b_hbm, out_hbm, a_v, b_v, bias_v):
        cid = lax.axis_index("c")
        sid = lax.axis_index("s")
        wid = sid * _NC + cid
        base = wid * _WORDS
        pltpu.sync_copy(p_hbm.at[0, pl.ds(base, _WORDS)], a_v)
        pltpu.sync_copy(p_hbm.at[1, pl.ds(base, _WORDS)], b_v)
        pltpu.sync_copy(b_hbm, bias_v)
        bias_vecs = [bias_v[pl.ds(j * 16, 16)] for j in range(_D // 16)]

        def row(r, carry):
            rb = r * _D
            for j in range(_D // 16):
                sl = pl.ds(rb + j * 16, 16)
                a_v[sl] = a_v[sl] + b_v[sl] + bias_vecs[j]
            return carry

        lax.fori_loop(0, _RROWS, row, 0)
        pltpu.sync_copy(a_v, out_hbm.at[pl.ds(base, _WORDS)])

    return k(parts_flat, bias)


def kernel(input, edge_index, edge_weight, W, b):
    edges = edge_index.reshape(2, _E // _CH, _CH)
    ew = edge_weight.reshape(_E // _CH, _CH)
    zeros = jnp.zeros((_NPAD, _D), jnp.float32)
    xw = _matmul(input, W)
    p1 = _hop(xw, edges, ew, zeros)
    agg = _reduce(p1.reshape(_NC, _NPAD * _D),
                  jnp.zeros((_D,), jnp.float32)).reshape(_NPAD, _D)
    p2 = _hop(agg, edges, ew, zeros)
    out = _reduce(p2.reshape(_NC, _NPAD * _D), b).reshape(_NPAD, _D)
    return out[:_N]
